# Initial kernel scaffold; baseline (speedup 1.0000x reference)
import jax
import jax.numpy as jnp
from jax.experimental import pallas as pl


def kernel(indices, values):
    raise NotImplementedError("write your pallas kernel here")


# speedup vs baseline: 3.2089x; 3.2089x over previous
import jax
import jax.numpy as jnp
from jax import lax
from jax.experimental import pallas as pl
from jax.experimental.pallas import tpu as pltpu
from jax.experimental.pallas import tpu_sc as plsc

N = 4096
NNZ_IN = 1677721
R_CHUNK = 256
NPASS = 8
ACC_WORDS = R_CHUNK * N
NOWN = 16
NTILES = 16
E_BLK = 2048
NNZ_PAD = ((NNZ_IN + NTILES * E_BLK - 1) // (NTILES * E_BLK)) * (NTILES * E_BLK)
T_PER_TILE = NNZ_PAD // NTILES
N_BLKS = T_PER_TILE // E_BLK
G_PER_BLK = E_BLK // 16
CAPB = 1024
CAPR = NNZ_PAD + NNZ_PAD // 4 + 65536
ZERO_W = 8192


def _body(rows_hbm, cols_hbm, vals_hbm, out_hbm, bins_i, bins_v,
          acc_spmem, zero_v, rbuf, cbuf, vbuf, bki, bkv, idx2, val2,
          cpbuf, cur_v, rc_smem):
  c = lax.axis_index("c")
  s = lax.axis_index("s")
  base = s * T_PER_TILE

  def zinit(g, carry):
    zero_v[pl.ds(g * 16, 16)] = jnp.zeros((16,), jnp.float32)
    return carry
  lax.fori_loop(0, ZERO_W // 16, zinit, jnp.int32(0))
  cur_v[pl.ds(0, 16)] = jnp.zeros((16,), jnp.int32)

  R = R_CHUNK
  share = R * N // NTILES
  slice_w = R * N // NOWN
  own_of = lambda x: lax.shift_right_logical(x, 16)

  def pass_body(p, pcarry):
    lo = p * (2 * R) + c * R

    @pl.when(s == 0)
    def _():
      for o in range(NOWN):
        rc_smem[o] = jnp.int32(0)
    def zfill(k, car):
      pltpu.sync_copy(zero_v, acc_spmem.at[pl.ds(s * share + k * ZERO_W, ZERO_W)])
      return car
    lax.fori_loop(0, share // ZERO_W, zfill, jnp.int32(0))
    plsc.subcore_barrier()

    def flush_bucket(o, fcarry):
      lanes = lax.iota(jnp.int32, 16)
      cur = cur_v[pl.ds(0, 16)]
      cnt = jnp.sum(jnp.where(lanes == o, cur, jnp.int32(0)))
      nr = lax.shift_right_logical(cnt + 127, 7)
      ceil = lax.shift_left(nr, 7)
      padi = jnp.full((16,), jnp.int32(1), jnp.int32) * (o * slice_w)
      padv = jnp.zeros((16,), jnp.float32)
      for gg in range(8):
        ppos = cnt + gg * 16 + lax.iota(jnp.int32, 16)
        pm = ppos < ceil
        plsc.store_scatter(bki, [o * CAPB + ppos], padi, mask=pm)
        plsc.store_scatter(bkv, [o * CAPB + ppos], padv, mask=pm)
      pl.delay(1000)
      off = plsc.fetch_and_add(rc_smem.at[o], ceil, subcore_id=0)
      rbase = pl.multiple_of((c * NOWN + o) * CAPR + off, 128)
      obase = pl.multiple_of(o * CAPB, 128)

      def cp(j, car):
        dsti = pl.multiple_of(rbase + j * 128, 128)
        srci = pl.multiple_of(obase + j * 128, 128)
        pltpu.sync_copy(bki.at[pl.ds(srci, 128)], bins_i.at[pl.ds(dsti, 128)])
        pltpu.sync_copy(bkv.at[pl.ds(srci, 128)], bins_v.at[pl.ds(dsti, 128)])
        return car
      lax.fori_loop(0, nr, cp, jnp.int32(0))
      cur_v[pl.ds(0, 16)] = jnp.where(lax.iota(jnp.int32, 16) == o,
                                      jnp.int32(0), cur)
      return fcarry

    def flush_all(carry):
      return lax.fori_loop(0, NOWN, flush_bucket, carry)

    def grp_body(g, carry):
      r = rbuf[pl.ds(g * 16, 16)]
      cc = cbuf[pl.ds(g * 16, 16)]
      v = vbuf[pl.ds(g * 16, 16)]
      m = (r >= lo) & (r < lo + R)
      lidx = (r - lo) * N + cc
      sig = 1.0 / (1.0 + jnp.exp(-v))
      own = own_of(lidx) & 15
      rank, _lastm = plsc.scan_count(own, mask=m)
      curg = plsc.load_gather(cur_v, [own])
      pos = curg + rank - 1
      plsc.store_scatter(bki, [own * CAPB + pos], lidx, mask=m)
      plsc.store_scatter(bkv, [own * CAPB + pos], sig, mask=m)
      plsc.addupdate_scatter(cur_v, [own], rank, mask=m & _lastm)
      cur = cur_v[pl.ds(0, 16)]
      anyfull = plsc.all_reduce_population_count(cur >= CAPB - 16)[0]
      return lax.cond(anyfull > 0, flush_all, lambda x: x, carry)

    def blk_body(b, carry):
      off = base + b * E_BLK
      pltpu.sync_copy(rows_hbm.at[pl.ds(off, E_BLK)], rbuf)
      pltpu.sync_copy(cols_hbm.at[pl.ds(off, E_BLK)], cbuf)
      pltpu.sync_copy(vals_hbm.at[pl.ds(off, E_BLK)], vbuf)
      return lax.fori_loop(0, G_PER_BLK, grp_body, carry)

    lax.fori_loop(0, N_BLKS, blk_body, jnp.int32(0))
    flush_all(jnp.int32(0))
    plsc.subcore_barrier()

    if True:
      cnt = plsc.fetch_and_add(rc_smem.at[s], jnp.int32(0), subcore_id=0)
      nr = lax.shift_right_logical(cnt, 7)
      rbase = pl.multiple_of((c * NOWN + s) * CAPR, 128)

      def crow(j, car):
        jj = j & 15
        srci = pl.multiple_of(rbase + j * 128, 128)
        pltpu.sync_copy(bins_i.at[pl.ds(srci, 128)], idx2.at[jj])
        pltpu.sync_copy(bins_v.at[pl.ds(srci, 128)], val2.at[jj])
        pltpu.sync_copy(val2.at[jj], acc_spmem.at[idx2.at[jj]], add=True)
        return car
      lax.fori_loop(0, nr, crow, jnp.int32(0))
    plsc.subcore_barrier()

    out_base = lo * N + s * share

    def cpout(k, car):
      pltpu.sync_copy(acc_spmem.at[pl.ds(s * share + k * ZERO_W, ZERO_W)], cpbuf)
      pltpu.sync_copy(cpbuf, out_hbm.at[pl.ds(out_base + k * ZERO_W, ZERO_W)])
      return car
    lax.fori_loop(0, share // ZERO_W, cpout, jnp.int32(0))
    return pcarry

  lax.fori_loop(0, NPASS, pass_body, jnp.int32(0))


@jax.jit
def _run(rows_p, cols_p, vals_p):
  mesh = plsc.VectorSubcoreMesh(core_axis_name="c", subcore_axis_name="s")
  f = pl.kernel(
      _body,
      out_type=jax.ShapeDtypeStruct((N * N,), jnp.float32),
      mesh=mesh,
      compiler_params=pltpu.CompilerParams(needs_layout_passes=False),
      scratch_types=[
          pltpu.MemorySpace.HBM((2 * NOWN * CAPR,), jnp.int32),
          pltpu.MemorySpace.HBM((2 * NOWN * CAPR,), jnp.float32),
          pltpu.VMEM_SHARED((ACC_WORDS,), jnp.float32),
          pltpu.VMEM((ZERO_W,), jnp.float32),
          pltpu.VMEM((E_BLK,), jnp.int32),
          pltpu.VMEM((E_BLK,), jnp.int32),
          pltpu.VMEM((E_BLK,), jnp.float32),
          pltpu.VMEM((NOWN * CAPB,), jnp.int32),
          pltpu.VMEM((NOWN * CAPB,), jnp.float32),
          pltpu.VMEM((16, 128), jnp.int32),
          pltpu.VMEM((16, 128), jnp.float32),
          pltpu.VMEM((ZERO_W,), jnp.float32),
          pltpu.VMEM((16,), jnp.int32),
          pltpu.SMEM((16,), jnp.int32),
      ],
  )
  return f(rows_p, cols_p, vals_p)


def kernel(indices, values):
  pad = NNZ_PAD - NNZ_IN
  rows = jnp.concatenate([indices[0].astype(jnp.int32),
                          jnp.full((pad,), N, jnp.int32)])
  cols = jnp.concatenate([indices[1].astype(jnp.int32),
                          jnp.zeros((pad,), jnp.int32)])
  vals = jnp.concatenate([values, jnp.zeros((pad,), jnp.float32)])
  out = _run(rows, cols, vals)
  return out.reshape(N, N)


# speedup vs baseline: 3.2904x; 1.0254x over previous
import jax
import jax.numpy as jnp
from jax import lax
from jax.experimental import pallas as pl
from jax.experimental.pallas import tpu as pltpu
from jax.experimental.pallas import tpu_sc as plsc

N = 4096
NNZ_IN = 1677721
R_CHUNK = 256
NPASS = 8
ACC_WORDS = R_CHUNK * N
NOWN = 16
NTILES = 16
E_BLK = 2048
NNZ_PAD = ((NNZ_IN + NTILES * E_BLK - 1) // (NTILES * E_BLK)) * (NTILES * E_BLK)
T_PER_TILE = NNZ_PAD // NTILES
N_BLKS = T_PER_TILE // E_BLK
G_PER_BLK = E_BLK // 16
CAPB = 1024
CAPR = NNZ_PAD + NNZ_PAD // 4 + 65536
ZERO_W = 8192


def _body(rows_hbm, cols_hbm, vals_hbm, out_hbm, bins_i, bins_v,
          acc_spmem, zero_v, rbuf, cbuf, vbuf, bki, bkv, idx2, val2,
          cpbuf, cur_v, rc_smem):
  c = lax.axis_index("c")
  s = lax.axis_index("s")
  base = s * T_PER_TILE

  def zinit(g, carry):
    zero_v[pl.ds(g * 16, 16)] = jnp.zeros((16,), jnp.float32)
    return carry
  lax.fori_loop(0, ZERO_W // 16, zinit, jnp.int32(0))
  cur_v[pl.ds(0, 16)] = jnp.zeros((16,), jnp.int32)

  R = R_CHUNK
  share = R * N // NTILES
  slice_w = R * N // NOWN
  own_of = lambda x: lax.shift_right_logical(x, 16)

  def pass_body(p, pcarry):
    lo = p * (2 * R) + c * R

    @pl.when(s == 0)
    def _():
      for o in range(NOWN):
        rc_smem[o] = jnp.int32(0)
    def zfill(k, car):
      pltpu.sync_copy(zero_v, acc_spmem.at[pl.ds(s * share + k * ZERO_W, ZERO_W)])
      return car
    lax.fori_loop(0, share // ZERO_W, zfill, jnp.int32(0))
    plsc.subcore_barrier()

    def flush_bucket(o, fcarry):
      lanes = lax.iota(jnp.int32, 16)
      cur = cur_v[pl.ds(0, 16)]
      cnt = jnp.sum(jnp.where(lanes == o, cur, jnp.int32(0)))
      nr = lax.shift_right_logical(cnt + 127, 7)
      ceil = lax.shift_left(nr, 7)
      padi = jnp.full((16,), jnp.int32(1), jnp.int32) * (o * slice_w)
      padv = jnp.zeros((16,), jnp.float32)
      for gg in range(8):
        ppos = cnt + gg * 16 + lax.iota(jnp.int32, 16)
        pm = ppos < ceil
        plsc.store_scatter(bki, [o * CAPB + ppos], padi, mask=pm)
        plsc.store_scatter(bkv, [o * CAPB + ppos], padv, mask=pm)
      pl.delay(200)
      off = plsc.fetch_and_add(rc_smem.at[o], ceil, subcore_id=0)
      rbase = pl.multiple_of((c * NOWN + o) * CAPR + off, 128)
      obase = pl.multiple_of(o * CAPB, 128)

      def cp(j, car):
        dsti = pl.multiple_of(rbase + j * 128, 128)
        srci = pl.multiple_of(obase + j * 128, 128)
        pltpu.sync_copy(bki.at[pl.ds(srci, 128)], bins_i.at[pl.ds(dsti, 128)])
        pltpu.sync_copy(bkv.at[pl.ds(srci, 128)], bins_v.at[pl.ds(dsti, 128)])
        return car
      lax.fori_loop(0, nr, cp, jnp.int32(0))
      cur_v[pl.ds(0, 16)] = jnp.where(lax.iota(jnp.int32, 16) == o,
                                      jnp.int32(0), cur)
      return fcarry

    def flush_all(carry):
      return lax.fori_loop(0, NOWN, flush_bucket, carry)

    def grp_body(g, carry):
      r = rbuf[pl.ds(g * 16, 16)]
      cc = cbuf[pl.ds(g * 16, 16)]
      v = vbuf[pl.ds(g * 16, 16)]
      m = (r >= lo) & (r < lo + R)
      lidx = (r - lo) * N + cc
      sig = 1.0 / (1.0 + jnp.exp(-v))
      own = own_of(lidx) & 15
      rank, _lastm = plsc.scan_count(own, mask=m)
      curg = plsc.load_gather(cur_v, [own])
      pos = curg + rank - 1
      plsc.store_scatter(bki, [own * CAPB + pos], lidx, mask=m)
      plsc.store_scatter(bkv, [own * CAPB + pos], sig, mask=m)
      plsc.addupdate_scatter(cur_v, [own], rank, mask=m & _lastm)
      cur = cur_v[pl.ds(0, 16)]
      anyfull = plsc.all_reduce_population_count(cur >= CAPB - 16)[0]
      return lax.cond(anyfull > 0, flush_all, lambda x: x, carry)

    def blk_body(b, carry):
      off = base + b * E_BLK
      pltpu.sync_copy(rows_hbm.at[pl.ds(off, E_BLK)], rbuf)
      pltpu.sync_copy(cols_hbm.at[pl.ds(off, E_BLK)], cbuf)
      pltpu.sync_copy(vals_hbm.at[pl.ds(off, E_BLK)], vbuf)
      return lax.fori_loop(0, G_PER_BLK, grp_body, carry)

    lax.fori_loop(0, N_BLKS, blk_body, jnp.int32(0))
    flush_all(jnp.int32(0))
    plsc.subcore_barrier()

    if True:
      cnt = plsc.fetch_and_add(rc_smem.at[s], jnp.int32(0), subcore_id=0)
      nr = lax.shift_right_logical(cnt, 7)
      rbase = pl.multiple_of((c * NOWN + s) * CAPR, 128)

      def crow(j, car):
        jj = j & 15
        srci = pl.multiple_of(rbase + j * 128, 128)
        pltpu.sync_copy(bins_i.at[pl.ds(srci, 128)], idx2.at[jj])
        pltpu.sync_copy(bins_v.at[pl.ds(srci, 128)], val2.at[jj])
        pltpu.sync_copy(val2.at[jj], acc_spmem.at[idx2.at[jj]], add=True)
        return car
      lax.fori_loop(0, nr, crow, jnp.int32(0))
    plsc.subcore_barrier()

    out_base = lo * N + s * share

    def cpout(k, car):
      pltpu.sync_copy(acc_spmem.at[pl.ds(s * share + k * ZERO_W, ZERO_W)], cpbuf)
      pltpu.sync_copy(cpbuf, out_hbm.at[pl.ds(out_base + k * ZERO_W, ZERO_W)])
      return car
    nvalid = jnp.maximum(jnp.int32(0),
                         jnp.minimum(jnp.int32(share), N * N - out_base))
    lax.fori_loop(0, lax.shift_right_logical(nvalid, 13), cpout, jnp.int32(0))
    return pcarry

  lax.fori_loop(0, NPASS, pass_body, jnp.int32(0))


@jax.jit
def _run(rows_p, cols_p, vals_p):
  mesh = plsc.VectorSubcoreMesh(core_axis_name="c", subcore_axis_name="s")
  f = pl.kernel(
      _body,
      out_type=jax.ShapeDtypeStruct((N * N,), jnp.float32),
      mesh=mesh,
      compiler_params=pltpu.CompilerParams(needs_layout_passes=False),
      scratch_types=[
          pltpu.MemorySpace.HBM((2 * NOWN * CAPR,), jnp.int32),
          pltpu.MemorySpace.HBM((2 * NOWN * CAPR,), jnp.float32),
          pltpu.VMEM_SHARED((ACC_WORDS,), jnp.float32),
          pltpu.VMEM((ZERO_W,), jnp.float32),
          pltpu.VMEM((E_BLK,), jnp.int32),
          pltpu.VMEM((E_BLK,), jnp.int32),
          pltpu.VMEM((E_BLK,), jnp.float32),
          pltpu.VMEM((NOWN * CAPB,), jnp.int32),
          pltpu.VMEM((NOWN * CAPB,), jnp.float32),
          pltpu.VMEM((16, 128), jnp.int32),
          pltpu.VMEM((16, 128), jnp.float32),
          pltpu.VMEM((ZERO_W,), jnp.float32),
          pltpu.VMEM((16,), jnp.int32),
          pltpu.SMEM((16,), jnp.int32),
      ],
  )
  return f(rows_p, cols_p, vals_p)


def kernel(indices, values):
  pad = NNZ_PAD - NNZ_IN
  rows = jnp.concatenate([indices[0].astype(jnp.int32),
                          jnp.full((pad,), N, jnp.int32)])
  cols = jnp.concatenate([indices[1].astype(jnp.int32),
                          jnp.zeros((pad,), jnp.int32)])
  vals = jnp.concatenate([values, jnp.zeros((pad,), jnp.float32)])
  out = _run(rows, cols, vals)
  return out.reshape(N, N)
